# Initial kernel scaffold; baseline (speedup 1.0000x reference)
import functools

import jax
import jax.numpy as jnp
from jax import lax
from jax.experimental import pallas as pl
from jax.experimental.pallas import tpu as pltpu
from jax.experimental.pallas import tpu_sc as plsc

N_NODES = 10000
N_EDGES = 320000
DIM = 128
NUM_RELS = 5

NUM_TILES = 32
EDGES_PER_TILE = N_EDGES // NUM_TILES
CHUNK = 80
CHUNKS_PER_TILE = EDGES_PER_TILE // CHUNK
ROWS_PER_TILE = N_NODES // 16


_HP_BLK = 1000


def _hp_body(h_ref, rv_ref, out_ref):
    out_ref[0] = jnp.maximum(h_ref[...] + rv_ref[...], 0.0)


def _build_hp(h, relvectors):
    return pl.pallas_call(
        _hp_body,
        grid=(NUM_RELS, N_NODES // _HP_BLK),
        in_specs=[
            pl.BlockSpec((_HP_BLK, DIM), lambda r, i: (i, 0)),
            pl.BlockSpec((1, DIM), lambda r, i: (r, 0)),
        ],
        out_specs=pl.BlockSpec((1, _HP_BLK, DIM), lambda r, i: (r, i, 0)),
        out_shape=jax.ShapeDtypeStruct((NUM_RELS, N_NODES, DIM), jnp.float32),
    )(h, relvectors)


def _sc_body(hp_hbm, src_hbm, dst_hbm, eid_hbm, z_hbm,
             acc0_hbm, acc1_hbm, cnt_hbm,
             src_v, dst_v, eid_v, gidx_v, rows_v, cnt_v, acc_sh, sem):
    c = lax.axis_index("c")
    s = lax.axis_index("s")
    wid = c * 16 + s

    pltpu.sync_copy(z_hbm, acc_sh.at[pl.ds(s * ROWS_PER_TILE, ROWS_PER_TILE)])

    zeros16 = jnp.zeros((16,), jnp.float32)
    ones16 = jnp.ones((16,), jnp.float32)

    @pl.loop(0, N_NODES, step=16)
    def _(i):
        cnt_v[pl.ds(i, 16)] = zeros16

    plsc.subcore_barrier()

    base_w = wid * EDGES_PER_TILE

    @pl.loop(0, CHUNKS_PER_TILE)
    def _(ci):
        base = base_w + ci * CHUNK
        pltpu.sync_copy(src_hbm.at[pl.ds(base, CHUNK)], src_v)
        pltpu.sync_copy(dst_hbm.at[pl.ds(base, CHUNK)], dst_v)
        pltpu.sync_copy(eid_hbm.at[pl.ds(base, CHUNK)], eid_v)

        @pl.loop(0, CHUNK, step=16)
        def _(j):
            sv = src_v[pl.ds(j, 16)]
            ev = eid_v[pl.ds(j, 16)]
            gidx_v[pl.ds(j, 16)] = ev * N_NODES + sv
            plsc.addupdate_scatter(cnt_v, [dst_v[pl.ds(j, 16)]], ones16)

        pltpu.async_copy(hp_hbm.at[gidx_v], rows_v, sem).wait()
        pltpu.sync_copy(rows_v, acc_sh.at[dst_v], add=True)

    plsc.subcore_barrier()

    pltpu.sync_copy(cnt_v, cnt_hbm.at[wid])
    row0 = s * ROWS_PER_TILE

    @pl.when(c == 0)
    def _():
        pltpu.sync_copy(acc_sh.at[pl.ds(row0, ROWS_PER_TILE)],
                        acc0_hbm.at[pl.ds(row0, ROWS_PER_TILE)])

    @pl.when(c == 1)
    def _():
        pltpu.sync_copy(acc_sh.at[pl.ds(row0, ROWS_PER_TILE)],
                        acc1_hbm.at[pl.ds(row0, ROWS_PER_TILE)])


def _sc_aggregate(hp, src, dst, eid, zrows):
    mesh = plsc.VectorSubcoreMesh(core_axis_name="c", subcore_axis_name="s")
    out_type = (
        jax.ShapeDtypeStruct((N_NODES, DIM), jnp.float32),
        jax.ShapeDtypeStruct((N_NODES, DIM), jnp.float32),
        jax.ShapeDtypeStruct((NUM_TILES, N_NODES), jnp.float32),
    )
    scratch = [
        pltpu.VMEM((CHUNK,), jnp.int32),
        pltpu.VMEM((CHUNK,), jnp.int32),
        pltpu.VMEM((CHUNK,), jnp.int32),
        pltpu.VMEM((CHUNK,), jnp.int32),
        pltpu.VMEM((CHUNK, DIM), jnp.float32),
        pltpu.VMEM((N_NODES,), jnp.float32),
        pltpu.VMEM_SHARED((N_NODES, DIM), jnp.float32),
        pltpu.SemaphoreType.DMA,
    ]
    fn = pl.kernel(_sc_body, out_type=out_type, mesh=mesh,
                   scratch_types=scratch)
    return fn(hp, src, dst, eid, zrows)


_FIN_BLK = 1000


def _fin_body(a0_ref, a1_ref, cnt_ref, w_ref, b_ref, out_ref):
    ssum = a0_ref[...] + a1_ref[...]
    cnt = jnp.sum(cnt_ref[...], axis=0)
    red = ssum / jnp.maximum(cnt, 1.0)[:, None]
    out_ref[...] = lax.dot_general(
        red, w_ref[...], (((1,), (1,)), ((), ())),
        preferred_element_type=jnp.float32) + b_ref[...]


def _finalize(acc0, acc1, cnts, W, b2):
    return pl.pallas_call(
        _fin_body,
        grid=(N_NODES // _FIN_BLK,),
        in_specs=[
            pl.BlockSpec((_FIN_BLK, DIM), lambda i: (i, 0)),
            pl.BlockSpec((_FIN_BLK, DIM), lambda i: (i, 0)),
            pl.BlockSpec((NUM_TILES, _FIN_BLK), lambda i: (0, i)),
            pl.BlockSpec((DIM, DIM), lambda i: (0, 0)),
            pl.BlockSpec((1, DIM), lambda i: (0, 0)),
        ],
        out_specs=pl.BlockSpec((_FIN_BLK, DIM), lambda i: (i, 0)),
        out_shape=jax.ShapeDtypeStruct((N_NODES, DIM), jnp.float32),
    )(acc0, acc1, cnts, W, b2)


@jax.jit
def kernel(h, edge_index, edge_id, W, b, relvectors):
    src = edge_index[0].astype(jnp.int32)
    dst = edge_index[1].astype(jnp.int32)
    eid = edge_id.astype(jnp.int32)
    hp = _build_hp(h, relvectors).reshape(NUM_RELS * N_NODES, DIM)
    zrows = jnp.zeros((ROWS_PER_TILE, DIM), jnp.float32)
    acc0, acc1, cnts = _sc_aggregate(hp, src, dst, eid, zrows)
    return _finalize(acc0, acc1, cnts, W, b.reshape(1, DIM))


# speedup vs baseline: 7.1702x; 7.1702x over previous
import dataclasses

import jax
import jax.numpy as jnp
from jax import lax
from jax.experimental import pallas as pl
from jax.experimental.pallas import tpu as pltpu
from jax.experimental.pallas import tpu_sc as plsc

N_NODES = 10000
N_EDGES = 320000
DIM = 128
NUM_RELS = 5

NPAD = 10240
NUM_TILES = 32
EDGES_PER_TILE = N_EDGES // NUM_TILES
CHUNK = 80
CHUNKS_PER_TILE = EDGES_PER_TILE // CHUNK
ROWS_PER_TILE = NPAD // 16


_HP_BLK = 1000


def _hp_body(h_ref, rv_ref, out_ref):
    hb = h_ref[...]
    for r in range(NUM_RELS):
        out_ref[r] = jnp.maximum(hb + rv_ref[r], 0.0)


def _build_hp(h, relvectors):
    return pl.pallas_call(
        _hp_body,
        grid=(N_NODES // _HP_BLK,),
        in_specs=[
            pl.BlockSpec((_HP_BLK, DIM), lambda i: (i, 0)),
            pl.BlockSpec((NUM_RELS, DIM), lambda i: (0, 0)),
        ],
        out_specs=pl.BlockSpec((NUM_RELS, _HP_BLK, DIM), lambda i: (0, i, 0)),
        out_shape=jax.ShapeDtypeStruct((NUM_RELS, N_NODES, DIM), jnp.float32),
    )(h, relvectors)


def _sc_body(hp_hbm, src_hbm, dst_hbm, eid_hbm, z_hbm,
             acc0_hbm, acc1_hbm, cnt_hbm,
             src_v, dst_v, eid_v, gidx_v, rows_v, cnt_v, acc_sh, sem):
    c = lax.axis_index("c")
    s = lax.axis_index("s")
    wid = c * 16 + s
    row0 = s * ROWS_PER_TILE

    pltpu.sync_copy(z_hbm, acc_sh.at[pl.ds(row0, ROWS_PER_TILE)])

    zeros16 = jnp.zeros((16,), jnp.float32)
    zidx16 = jnp.zeros((16,), jnp.int32)
    ones16 = jnp.ones((16,), jnp.float32)

    @pl.loop(0, NPAD, step=16)
    def _(i):
        cnt_v[0, pl.ds(i, 16)] = zeros16

    plsc.subcore_barrier()

    base_w = wid * EDGES_PER_TILE

    @pl.loop(0, CHUNKS_PER_TILE)
    def _(ci):
        base = base_w + ci * CHUNK
        pltpu.sync_copy(src_hbm.at[pl.ds(base, CHUNK)], src_v)
        pltpu.sync_copy(dst_hbm.at[pl.ds(base, CHUNK)], dst_v)
        pltpu.sync_copy(eid_hbm.at[pl.ds(base, CHUNK)], eid_v)

        @pl.loop(0, CHUNK, step=16)
        def _(j):
            sv = src_v[pl.ds(j, 16)]
            ev = eid_v[pl.ds(j, 16)]
            gidx_v[pl.ds(j, 16)] = ev * N_NODES + sv
            plsc.addupdate_scatter(cnt_v, [zidx16, dst_v[pl.ds(j, 16)]],
                                   ones16)

        pltpu.async_copy(hp_hbm.at[gidx_v], rows_v, sem).wait()
        pltpu.sync_copy(rows_v, acc_sh.at[dst_v], add=True)

    plsc.subcore_barrier()

    pltpu.sync_copy(cnt_v, cnt_hbm.at[wid])

    @pl.when(c == 0)
    def _():
        pltpu.sync_copy(acc_sh.at[pl.ds(row0, ROWS_PER_TILE)],
                        acc0_hbm.at[pl.ds(row0, ROWS_PER_TILE)])

    @pl.when(c == 1)
    def _():
        pltpu.sync_copy(acc_sh.at[pl.ds(row0, ROWS_PER_TILE)],
                        acc1_hbm.at[pl.ds(row0, ROWS_PER_TILE)])


def _sc_aggregate(hp, src, dst, eid, zrows):
    mesh = plsc.VectorSubcoreMesh(core_axis_name="c", subcore_axis_name="s")
    out_type = (
        jax.ShapeDtypeStruct((NPAD, DIM), jnp.float32),
        jax.ShapeDtypeStruct((NPAD, DIM), jnp.float32),
        jax.ShapeDtypeStruct((NUM_TILES, 1, NPAD), jnp.float32),
    )
    scratch = [
        pltpu.VMEM((CHUNK,), jnp.int32),
        pltpu.VMEM((CHUNK,), jnp.int32),
        pltpu.VMEM((CHUNK,), jnp.int32),
        pltpu.VMEM((CHUNK,), jnp.int32),
        pltpu.VMEM((CHUNK, DIM), jnp.float32),
        pltpu.VMEM((1, NPAD), jnp.float32),
        pltpu.VMEM_SHARED((NPAD, DIM), jnp.float32),
        pltpu.SemaphoreType.DMA,
    ]
    cp = pltpu.CompilerParams()
    if "needs_layout_passes" in pltpu.CompilerParams.__dataclass_fields__:
        cp = dataclasses.replace(cp, needs_layout_passes=False)
    fn = pl.kernel(_sc_body, out_type=out_type, mesh=mesh,
                   scratch_types=scratch, compiler_params=cp)
    return fn(hp, src, dst, eid, zrows)


_FIN_BLK = 1024


def _fin_body(a0_ref, a1_ref, cnt_ref, w_ref, b_ref, out_ref):
    ssum = a0_ref[...] + a1_ref[...]
    csum = jnp.sum(cnt_ref[...], axis=0)
    eye = (lax.broadcasted_iota(jnp.int32, (DIM, DIM), 0)
           == lax.broadcasted_iota(jnp.int32, (DIM, DIM), 1)
           ).astype(jnp.float32)
    ct = lax.dot_general(eye, csum, (((1,), (1,)), ((), ())),
                         preferred_element_type=jnp.float32)
    pieces = []
    for k in range(_FIN_BLK // DIM):
        col = jnp.maximum(ct[:, k:k + 1], 1.0)
        pieces.append(ssum[k * DIM:(k + 1) * DIM, :] / col)
    red = jnp.concatenate(pieces, axis=0)
    out_ref[...] = lax.dot_general(
        red, w_ref[...], (((1,), (1,)), ((), ())),
        preferred_element_type=jnp.float32) + b_ref[...]


def _finalize(acc0, acc1, cnts, W, b2):
    return pl.pallas_call(
        _fin_body,
        grid=(NPAD // _FIN_BLK,),
        in_specs=[
            pl.BlockSpec((_FIN_BLK, DIM), lambda g: (g, 0)),
            pl.BlockSpec((_FIN_BLK, DIM), lambda g: (g, 0)),
            pl.BlockSpec((NUM_TILES, _FIN_BLK // DIM, DIM), lambda g: (0, g, 0)),
            pl.BlockSpec((DIM, DIM), lambda g: (0, 0)),
            pl.BlockSpec((1, DIM), lambda g: (0, 0)),
        ],
        out_specs=pl.BlockSpec((_FIN_BLK, DIM), lambda g: (g, 0)),
        out_shape=jax.ShapeDtypeStruct((NPAD, DIM), jnp.float32),
    )(acc0, acc1, cnts, W, b2)


@jax.jit
def kernel(h, edge_index, edge_id, W, b, relvectors):
    src = edge_index[0].astype(jnp.int32)
    dst = edge_index[1].astype(jnp.int32)
    eid = edge_id.astype(jnp.int32)
    hp = _build_hp(h, relvectors).reshape(NUM_RELS * N_NODES, DIM)
    zrows = jnp.zeros((ROWS_PER_TILE, DIM), jnp.float32)
    acc0, acc1, cnts = _sc_aggregate(hp, src, dst, eid, zrows)
    cnts = cnts.reshape(NUM_TILES, NPAD // DIM, DIM)
    out = _finalize(acc0, acc1, cnts, W, b.reshape(1, DIM))
    return out[:N_NODES]
